# Initial kernel scaffold; baseline (speedup 1.0000x reference)
import jax
import jax.numpy as jnp
from jax.experimental import pallas as pl


def kernel(idx, child_tokens, counts, W, b_lin):
    raise NotImplementedError("write your pallas kernel here")


# speedup vs baseline: 9.0709x; 9.0709x over previous
import jax
import jax.numpy as jnp
from jax import lax
from jax.experimental import pallas as pl
from jax.experimental.pallas import tpu as pltpu
from jax.experimental.pallas import tpu_sc as plsc

_VOCAB = 100000
_DEPTH = 4
_NCHILD = 64
_B, _T = 4, 32
_ROWS = _B * _T
_UPD = _DEPTH * _NCHILD
_NW = 32
_ROWS_PER_W = _ROWS // _NW
_FILL_UNROLL = 25
_FILL_STEPS = _VOCAB // (16 * _FILL_UNROLL)


def _sc_body(tok_hbm, cnt_hbm, w_hbm, b_hbm, out_hbm,
             row_v, tok_v, cnt_v, w_v, b_v):
    wid = lax.axis_index("s") * 2 + lax.axis_index("c")
    pltpu.sync_copy(w_hbm, w_v)
    pltpu.sync_copy(b_hbm, b_v)
    bv = b_v[...]
    lane = lax.iota(jnp.int32, 16)

    def _fill(i, c):
        base = i * (16 * _FILL_UNROLL)
        for u in range(_FILL_UNROLL):
            row_v[pl.ds(base + u * 16, 16)] = bv
        return c

    for r in range(_ROWS_PER_W):
        row = wid * _ROWS_PER_W + r
        pltpu.sync_copy(tok_hbm.at[row], tok_v)
        pltpu.sync_copy(cnt_hbm.at[row], cnt_v)
        lax.fori_loop(0, _FILL_STEPS, _fill, 0)
        for d in range(_DEPTH):
            dbase = d * _NCHILD
            toks = [tok_v[pl.ds(dbase + c * 16, 16)] for c in range(4)]
            cnts = [cnt_v[pl.ds(dbase + c * 16, 16)] for c in range(4)]
            olds = [plsc.load_gather(row_v, [toks[c]]) for c in range(4)]
            wd = w_v[d]
            news = [olds[c] + wd * cnts[c] for c in range(4)]
            for c in range(4):
                dup = lane < 0
                for j in range(1, 16):
                    bc = plsc.load_gather(
                        tok_v, [jnp.full((16,), dbase + c * 16 + j, jnp.int32)])
                    dup = jnp.logical_or(
                        dup, jnp.logical_and(toks[c] == bc, lane < j))
                plsc.store_scatter(row_v, [toks[c]], news[c],
                                   mask=jnp.logical_not(dup))
        pltpu.sync_copy(row_v, out_hbm.at[row])


def _make_call():
    mesh = plsc.VectorSubcoreMesh(core_axis_name="c", subcore_axis_name="s")
    return pl.kernel(
        _sc_body,
        out_type=jax.ShapeDtypeStruct((_ROWS, _VOCAB), jnp.float32),
        mesh=mesh,
        compiler_params=pltpu.CompilerParams(needs_layout_passes=False),
        scratch_types=[
            pltpu.VMEM((_VOCAB,), jnp.float32),
            pltpu.VMEM((_UPD,), jnp.int32),
            pltpu.VMEM((_UPD,), jnp.float32),
            pltpu.VMEM((_DEPTH, 16), jnp.float32),
            pltpu.VMEM((16,), jnp.float32),
        ],
    )


def kernel(idx, child_tokens, counts, W, b_lin):
    del idx
    tok = child_tokens.reshape(_ROWS, _UPD)
    cnt = counts.reshape(_ROWS, _UPD).astype(jnp.float32)
    wb = jnp.broadcast_to(W.reshape(_DEPTH, 1).astype(jnp.float32), (_DEPTH, 16))
    bb = jnp.broadcast_to(b_lin.reshape(1).astype(jnp.float32), (16,))
    out = _make_call()(tok, cnt, wb, bb)
    return out.reshape(_B, _T, _VOCAB)
